# Initial kernel scaffold; baseline (speedup 1.0000x reference)
import jax
import jax.numpy as jnp
from jax.experimental import pallas as pl


def kernel(node_feat_input, adjacency_input, indices, weight, bias):
    raise NotImplementedError("write your pallas kernel here")


# speedup vs baseline: 6.3521x; 6.3521x over previous
import jax
import jax.numpy as jnp
from jax import lax
from jax.experimental import pallas as pl
from jax.experimental.pallas import tpu as pltpu
from jax.experimental.pallas import tpu_sc as plsc

N = 10000
D = 128
E = 320000
NC = 2
NS = 16
NW = NC * NS
EPW = E // NW
K = 80
NCHUNK = EPW // K
NP = 10240
RPS = NP // NS


def _sc_aggregate_kernel(x_hbm, src_hbm, dst_hbm, featp_hbm, degp_hbm,
                         isrc_v, idst_v, rows_v, ones_v, zbuf_v, zdeg_v,
                         accf_s, accd_s, gsem):
  c = lax.axis_index("c")
  s = lax.axis_index("s")
  w = c * NS + s

  def _init_row(i, carry):
    ones_v[pl.ds(i * 16, 16)] = jnp.ones((16,), jnp.float32)
    return carry
  lax.fori_loop(0, K // 16, _init_row, 0)

  def _zero1_row(i, carry):
    zdeg_v[pl.ds(i * 16, 16)] = jnp.zeros((16,), jnp.float32)
    return carry
  lax.fori_loop(0, RPS // 16, _zero1_row, 0)

  def _zero_row(i, carry):
    for j in range(8):
      zbuf_v[i, pl.ds(j * 16, 16)] = jnp.zeros((16,), jnp.float32)
    return carry
  lax.fori_loop(0, 128, _zero_row, 0)

  rowbase = s * RPS
  for t in range(RPS // 128):
    pltpu.sync_copy(zbuf_v, accf_s.at[pl.ds(rowbase + t * 128, 128)])
  pltpu.sync_copy(zdeg_v, accd_s.at[pl.ds(rowbase, RPS)])
  plsc.subcore_barrier()

  ebase = w * EPW

  def _chunk(i, carry):
    off = ebase + i * K
    pltpu.sync_copy(src_hbm.at[pl.ds(off, K)], isrc_v)
    pltpu.sync_copy(dst_hbm.at[pl.ds(off, K)], idst_v)
    pltpu.async_copy(x_hbm.at[isrc_v], rows_v, gsem).wait()
    pltpu.sync_copy(rows_v, accf_s.at[idst_v], add=True)
    pltpu.sync_copy(ones_v, accd_s.at[idst_v], add=True)
    return carry
  lax.fori_loop(0, NCHUNK, _chunk, 0)

  plsc.subcore_barrier()

  pltpu.sync_copy(accf_s.at[pl.ds(rowbase, RPS)],
                  featp_hbm.at[c, pl.ds(rowbase, RPS)])
  pltpu.sync_copy(accd_s.at[pl.ds(rowbase, RPS)],
                  degp_hbm.at[pl.ds(c * NP + rowbase, RPS)])


@jax.jit
def _sc_aggregate(x, src, dst):
  mesh = plsc.VectorSubcoreMesh(core_axis_name="c", subcore_axis_name="s")
  return pl.kernel(
      _sc_aggregate_kernel,
      out_type=[
          jax.ShapeDtypeStruct((NC, NP, D), jnp.float32),
          jax.ShapeDtypeStruct((NC * NP,), jnp.float32),
      ],
      mesh=mesh,
      scratch_types=[
          pltpu.VMEM((K,), jnp.int32),
          pltpu.VMEM((K,), jnp.int32),
          pltpu.VMEM((K, D), jnp.float32),
          pltpu.VMEM((K,), jnp.float32),
          pltpu.VMEM((128, D), jnp.float32),
          pltpu.VMEM((RPS,), jnp.float32),
          pltpu.VMEM_SHARED((NP, D), jnp.float32),
          pltpu.VMEM_SHARED((NP,), jnp.float32),
          pltpu.SemaphoreType.DMA,
      ],
  )(x, src, dst)


BN = 1024


def _tc_finish_kernel(x_ref, fp_ref, dp_ref, w_ref, b_ref, o_ref):
  ssum = fp_ref[0] + fp_ref[1]
  deg = (dp_ref[0] + dp_ref[1]).reshape(BN, 1)
  h = ssum * (1.0 / deg)
  t = (jnp.dot(x_ref[...], w_ref[pl.ds(0, D)],
               preferred_element_type=jnp.float32)
       + jnp.dot(h, w_ref[pl.ds(D, D)],
                 preferred_element_type=jnp.float32)
       + b_ref[...])
  o_ref[...] = jax.nn.sigmoid(t)


@jax.jit
def _tc_finish(x, featp, degp, weight, bias):
  grid = ((N + BN - 1) // BN,)
  return pl.pallas_call(
      _tc_finish_kernel,
      grid=grid,
      in_specs=[
          pl.BlockSpec((BN, D), lambda i: (i, 0)),
          pl.BlockSpec((NC, BN, D), lambda i: (0, i, 0)),
          pl.BlockSpec((NC, BN), lambda i: (0, i)),
          pl.BlockSpec((2 * D, D), lambda i: (0, 0)),
          pl.BlockSpec((1, D), lambda i: (0, 0)),
      ],
      out_specs=pl.BlockSpec((BN, D), lambda i: (i, 0)),
      out_shape=jax.ShapeDtypeStruct((N, D), jnp.float32),
  )(x, featp, degp.reshape(NC, NP), weight, bias.reshape(1, D))


@jax.jit
def kernel(node_feat_input, adjacency_input, indices, weight, bias):
  del indices
  dst = adjacency_input[:, 0]
  src = adjacency_input[:, 1]
  featp, degp = _sc_aggregate(node_feat_input, src, dst)
  return _tc_finish(node_feat_input, featp, degp, weight, bias)
